# baseline (speedup 1.0000x reference)
import functools

import jax
import jax.numpy as jnp
from jax.experimental import pallas as pl
from jax.experimental.pallas import tpu as pltpu

_B = 8
_K = 1024
_V = 100000
_C = 2048
_N = (_V + _C - 1) // _C


def _argmax_body(x_ref, w_ref, b_ref, g_ref, idx_ref, bv_ref, bi_ref):
    i = pl.program_id(0)
    logits = jnp.dot(x_ref[...], w_ref[...], preferred_element_type=jnp.float32)
    logits = logits + b_ref[...] + g_ref[...]
    cols = i * _C + jax.lax.broadcasted_iota(jnp.int32, logits.shape, 1)
    logits = jnp.where(cols < _V, logits, -jnp.inf)
    m = jnp.max(logits, axis=1, keepdims=True)
    cand = jnp.min(
        jnp.where(logits == m, cols, jnp.int32(2**31 - 1)), axis=1, keepdims=True
    )

    @pl.when(i == 0)
    def _():
        bv_ref[...] = m
        bi_ref[...] = cand

    @pl.when(i > 0)
    def _():
        bv = bv_ref[...]
        upd = m > bv
        bv_ref[...] = jnp.where(upd, m, bv)
        bi_ref[...] = jnp.where(upd, cand, bi_ref[...])

    @pl.when(i == _N - 1)
    def _():
        idx_ref[...] = bi_ref[...]


def _onehot_body(idx_ref, s_ref, sg_ref):
    i = pl.program_id(0)
    cols = i * _C + jax.lax.broadcasted_iota(jnp.int32, s_ref.shape, 1)
    oh = (cols == idx_ref[...]).astype(jnp.float32)
    s_ref[...] = oh
    sg_ref[...] = oh


@functools.partial(jax.jit, static_argnames=())
def kernel(x, W, b):
    g = jax.random.gumbel(jax.random.key(42), (_B, _V), dtype=jnp.float32)
    b2 = b.reshape(1, _V)
    idx = pl.pallas_call(
        _argmax_body,
        grid=(_N,),
        in_specs=[
            pl.BlockSpec((_B, _K), lambda i: (0, 0)),
            pl.BlockSpec((_K, _C), lambda i: (0, i)),
            pl.BlockSpec((1, _C), lambda i: (0, i)),
            pl.BlockSpec((_B, _C), lambda i: (0, i)),
        ],
        out_specs=pl.BlockSpec((_B, 1), lambda i: (0, 0)),
        out_shape=jax.ShapeDtypeStruct((_B, 1), jnp.int32),
        scratch_shapes=[
            pltpu.VMEM((_B, 1), jnp.float32),
            pltpu.VMEM((_B, 1), jnp.int32),
        ],
    )(x, W, b2, g)
    sample, sample_grad = pl.pallas_call(
        _onehot_body,
        grid=(_N,),
        in_specs=[pl.BlockSpec((_B, 1), lambda i: (0, 0))],
        out_specs=[
            pl.BlockSpec((_B, _C), lambda i: (0, i)),
            pl.BlockSpec((_B, _C), lambda i: (0, i)),
        ],
        out_shape=[
            jax.ShapeDtypeStruct((_B, _V), jnp.float32),
            jax.ShapeDtypeStruct((_B, _V), jnp.float32),
        ],
    )(idx)
    return (sample, sample_grad)
